# Initial kernel scaffold; baseline (speedup 1.0000x reference)
import jax
import jax.numpy as jnp
from jax.experimental import pallas as pl


def kernel(ui_pair, preference_index, ui_table, attri_table, bias):
    raise NotImplementedError("write your pallas kernel here")


# speedup vs baseline: 1.5248x; 1.5248x over previous
import functools

import jax
import jax.numpy as jnp
from jax import lax
from jax.experimental import pallas as pl
from jax.experimental.pallas import tpu as pltpu
from jax.experimental.pallas import tpu_sc as plsc

EMB = 64
L = 20
NROWS = 2 + L
NW = 32
LANES = 16
NBLK = EMB // LANES


def _fm_kernel(B, C):
  rows_per_w = B // NW
  n_chunks = rows_per_w // C
  mesh = plsc.VectorSubcoreMesh(core_axis_name="c", subcore_axis_name="s")

  @functools.partial(
      pl.kernel,
      out_type=(
          jax.ShapeDtypeStruct((B * NROWS * EMB,), jnp.float32),
          jax.ShapeDtypeStruct((B,), jnp.float32),
      ),
      mesh=mesh,
      compiler_params=pltpu.CompilerParams(
          needs_layout_passes=False, use_tc_tiling_on_sc=False),
      scratch_types=[
          pltpu.VMEM((2 * C,), jnp.int32),
          pltpu.VMEM((L * C,), jnp.int32),
          pltpu.VMEM((2 * C, EMB), jnp.float32),
          pltpu.VMEM((L * C, EMB), jnp.float32),
          pltpu.VMEM((C * NROWS * EMB,), jnp.float32),
          pltpu.VMEM((C,), jnp.float32),
          pltpu.VMEM((C * LANES,), jnp.float32),
          pltpu.VMEM((LANES,), jnp.float32),
          pltpu.SemaphoreType.DMA,
      ],
  )
  def k(ui_idx_h, pref_idx_h, ui_table_h, attri_table_h, bias_h,
        fm_out, res_out,
        uidx_v, pidx_v, ui_sep, p_sep, fm_buf, res_buf, t_buf, bias_v, sem):
    wid = lax.axis_index("s") * 2 + lax.axis_index("c")
    pltpu.sync_copy(bias_h, bias_v)
    lane = lax.iota(jnp.int32, LANES)
    last_lane = lane == (LANES - 1)

    def chunk_body(ci, carry):
      base = wid * rows_per_w + ci * C
      pltpu.sync_copy(ui_idx_h.at[pl.ds(base * 2, 2 * C)], uidx_v)
      pltpu.sync_copy(pref_idx_h.at[pl.ds(base * L, L * C)], pidx_v)
      copies = [pltpu.async_copy(ui_table_h.at[uidx_v], ui_sep, sem)]
      for g in range(L * C // 128):
        copies.append(pltpu.async_copy(
            attri_table_h.at[pidx_v.at[pl.ds(g * 128, 128)]],
            p_sep.at[pl.ds(g * 128, 128)], sem))
      for cp in copies:
        cp.wait()

      def row_body(i, rcarry):
        t = jnp.zeros((LANES,), jnp.float32)
        for kb in range(NBLK):
          sl = pl.ds(kb * LANES, LANES)
          u0 = ui_sep[2 * i, sl]
          u1 = ui_sep[2 * i + 1, sl]
          off = i * (NROWS * EMB) + kb * LANES
          fm_buf[pl.ds(off, LANES)] = u0
          fm_buf[pl.ds(off + EMB, LANES)] = u1
          acc = jnp.zeros((LANES,), jnp.float32)
          for j in range(L):
            r = p_sep[i * L + j, sl]
            fm_buf[pl.ds(off + (2 + j) * EMB, LANES)] = r
            acc = acc + r
          t = t + u0 * u1 + (u0 + u1) * acc
        t_buf[pl.ds(i * LANES, LANES)] = t
        return rcarry

      lax.fori_loop(0, C, row_body, 0)
      for g in range(C // LANES):
        rsum = jnp.zeros((LANES,), jnp.float32)
        col0 = (jnp.int32(g * LANES) + lane) * LANES
        for d in range(LANES):
          rsum = rsum + plsc.load_gather(t_buf, [col0 + d])
        res_buf[pl.ds(g * LANES, LANES)] = rsum + bias_v[...]
      pltpu.sync_copy(fm_buf, fm_out.at[pl.ds(base * NROWS * EMB, C * NROWS * EMB)])
      pltpu.sync_copy(res_buf, res_out.at[pl.ds(base, C)])
      return carry

    lax.fori_loop(0, n_chunks, chunk_body, 0)

  return k


def kernel(ui_pair, preference_index, ui_table, attri_table, bias):
  B = ui_pair.shape[0]
  C = 32
  ui_idx = ui_pair.reshape(-1)
  pref_idx = preference_index.reshape(-1)
  bias16 = jnp.broadcast_to(bias, (LANES,))
  fm, res = _fm_kernel(B, C)(ui_idx, pref_idx, ui_table, attri_table, bias16)
  return (res.reshape(B, 1), fm.reshape(B, NROWS, EMB))


# speedup vs baseline: 1.6747x; 1.0983x over previous
import functools

import jax
import jax.numpy as jnp
from jax import lax
from jax.experimental import pallas as pl
from jax.experimental.pallas import tpu as pltpu
from jax.experimental.pallas import tpu_sc as plsc

EMB = 64
L = 20
NROWS = 2 + L
NW = 32
LANES = 16
NBLK = EMB // LANES


def _fm_kernel(B, C):
  rows_per_w = B // NW
  n_chunks = rows_per_w // C
  mesh = plsc.VectorSubcoreMesh(core_axis_name="c", subcore_axis_name="s")

  @functools.partial(
      pl.kernel,
      out_type=(
          jax.ShapeDtypeStruct((B * NROWS, EMB), jnp.float32),
          jax.ShapeDtypeStruct((B,), jnp.float32),
      ),
      mesh=mesh,
      compiler_params=pltpu.CompilerParams(
          needs_layout_passes=False, use_tc_tiling_on_sc=False),
      scratch_types=[
          pltpu.VMEM((2 * C,), jnp.int32),
          pltpu.VMEM((2 * C,), jnp.int32),
          pltpu.VMEM((C, L), jnp.int32),
          pltpu.VMEM((C, L), jnp.int32),
          pltpu.VMEM((2 * C, EMB), jnp.float32),
          pltpu.VMEM((2 * C, EMB), jnp.float32),
          pltpu.VMEM((C * NROWS, EMB), jnp.float32),
          pltpu.VMEM((C * NROWS, EMB), jnp.float32),
          pltpu.VMEM((rows_per_w,), jnp.float32),
          pltpu.VMEM((C * LANES,), jnp.float32),
          pltpu.VMEM((LANES,), jnp.float32),
          pltpu.SemaphoreType.DMA,
          pltpu.SemaphoreType.DMA,
      ],
  )
  def k(ui_idx_h, pref_idx_h, ui_table_h, attri_table_h, bias_h,
        fm_out, res_out,
        uidx_a, uidx_b, pidx_a, pidx_b, ui_a, ui_b, fm_a, fm_b,
        res_buf, t_buf, bias_v, gsem, osem):
    cid = lax.axis_index("c")
    sid = lax.axis_index("s")
    wid = sid * 2 + cid
    wbase = wid * rows_per_w
    pltpu.sync_copy(bias_h, bias_v)
    lane = lax.iota(jnp.int32, LANES)

    def gathers(ci, uidx_v, pidx_v, ui_sep, fm_buf):
      base = wbase + ci * C
      pltpu.sync_copy(ui_idx_h.at[pl.ds(base * 2, 2 * C)], uidx_v)
      pltpu.sync_copy(pref_idx_h.at[pl.ds(base, C)], pidx_v)
      copies = [pltpu.async_copy(ui_table_h.at[uidx_v], ui_sep, gsem)]
      for i in range(C):
        copies.append(pltpu.async_copy(
            attri_table_h.at[pidx_v.at[i]],
            fm_buf.at[pl.ds(i * NROWS + 2, L)], gsem))
      return copies

    def compute(ci, ui_sep, fm_buf):
      def row_body(i, rcarry):
        t = jnp.zeros((LANES,), jnp.float32)
        for kb in range(NBLK):
          sl = pl.ds(kb * LANES, LANES)
          u0 = ui_sep[2 * i, sl]
          u1 = ui_sep[2 * i + 1, sl]
          fm_buf[i * NROWS, sl] = u0
          fm_buf[i * NROWS + 1, sl] = u1
          acc = jnp.zeros((LANES,), jnp.float32)
          for j in range(L):
            acc = acc + fm_buf[i * NROWS + 2 + j, sl]
          t = t + u0 * u1 + (u0 + u1) * acc
        t_buf[pl.ds(i * LANES, LANES)] = t
        return rcarry

      lax.fori_loop(0, C, row_body, 0)
      for g in range(C // LANES):
        rsum = jnp.zeros((LANES,), jnp.float32)
        col0 = (jnp.int32(g * LANES) + lane) * LANES
        for d in range(LANES):
          rsum = rsum + plsc.load_gather(t_buf, [col0 + d])
        res_buf[pl.ds(ci * C + g * LANES, LANES)] = rsum + bias_v[...]

    def out_dma(ci, fm_buf):
      return pltpu.make_async_copy(
          fm_buf, fm_out.at[pl.ds((wbase + ci * C) * NROWS, C * NROWS)], osem)

    def process(ci, uidx_v, pidx_v, ui_sep, fm_buf):
      copies = gathers(ci, uidx_v, pidx_v, ui_sep, fm_buf)
      for cp in copies:
        cp.wait()
      compute(ci, ui_sep, fm_buf)
      out_dma(ci, fm_buf).start()

    def pair_body(kk, carry):
      process(2 * kk, uidx_a, pidx_a, ui_a, fm_a)
      process(2 * kk + 1, uidx_b, pidx_b, ui_b, fm_b)
      out_dma(2 * kk, fm_a).wait()
      out_dma(2 * kk + 1, fm_b).wait()
      return carry

    lax.fori_loop(0, n_chunks // 2, pair_body, 0)
    pltpu.sync_copy(res_buf, res_out.at[pl.ds(wbase, rows_per_w)])

  return k


def kernel(ui_pair, preference_index, ui_table, attri_table, bias):
  B = ui_pair.shape[0]
  C = 32
  ui_idx = ui_pair.reshape(-1)
  bias16 = jnp.broadcast_to(bias, (LANES,))
  fm, res = _fm_kernel(B, C)(
      ui_idx, preference_index, ui_table, attri_table, bias16)
  return (res.reshape(B, 1), fm.reshape(B, NROWS, EMB))
